# baseline (speedup 1.0000x reference)
import functools

import jax
import jax.numpy as jnp
from jax import lax
from jax.experimental import pallas as pl
from jax.experimental.pallas import tpu as pltpu
from jax.experimental.pallas import tpu_sc as plsc

_B = 16384
_D = 32
_NC = 2
_NS = 16
_NW = _NC * _NS
_BW = _B // _NW
_CH = 128
_NCH = _BW // _CH

_mesh = plsc.VectorSubcoreMesh(core_axis_name="c", subcore_axis_name="s")


@functools.partial(
    pl.kernel,
    mesh=_mesh,
    compiler_params=pltpu.CompilerParams(needs_layout_passes=False,
                                         use_tc_tiling_on_sc=False),
    out_type=jax.ShapeDtypeStruct((_B,), jnp.float32),
    scratch_types=[
        pltpu.VMEM((_BW,), jnp.int32),
        pltpu.VMEM((_BW,), jnp.int32),
        pltpu.VMEM((_BW, _D), jnp.float32),
        pltpu.VMEM((_BW, _D), jnp.float32),
        pltpu.VMEM((80,), jnp.float32),
        pltpu.VMEM((_BW,), jnp.float32),
        pltpu.SemaphoreType.DMA,
    ],
)
def _embed_dot(users_hbm, movies_hbm, utab_hbm, mtab_hbm, wb_hbm, out_hbm,
               uidx, midx, urows, mrows, wv, outv, sem):
    wid = lax.axis_index("s") * _NC + lax.axis_index("c")
    base = wid * _BW

    pltpu.sync_copy(users_hbm.at[pl.ds(base, _BW)], uidx)
    pltpu.sync_copy(movies_hbm.at[pl.ds(base, _BW)], midx)
    pltpu.sync_copy(wb_hbm, wv)

    copies = []
    for c in range(_NCH):
        sl = pl.ds(c * _CH, _CH)
        copies.append(pltpu.async_copy(utab_hbm.at[uidx.at[sl]], urows.at[sl], sem))
        copies.append(pltpu.async_copy(mtab_hbm.at[midx.at[sl]], mrows.at[sl], sem))
    for cp in copies:
        cp.wait()

    wu1 = wv[pl.ds(0, 16)]
    wu2 = wv[pl.ds(16, 16)]
    wm1 = wv[pl.ds(32, 16)]
    wm2 = wv[pl.ds(48, 16)]
    bias = wv[pl.ds(64, 16)][0]
    lanes = lax.iota(jnp.int32, 16)

    def group(g, _):
        acc = jnp.zeros((16,), jnp.float32)
        for j in range(16):
            i = g * 16 + j
            p = (urows[i, pl.ds(0, 16)] * wu1
                 + urows[i, pl.ds(16, 16)] * wu2
                 + mrows[i, pl.ds(0, 16)] * wm1
                 + mrows[i, pl.ds(16, 16)] * wm2)
            acc = jnp.where(lanes == j, jnp.sum(p), acc)
        outv[pl.ds(g * 16, 16)] = acc + bias
        return 0

    lax.fori_loop(0, _BW // 16, group, 0)

    pltpu.sync_copy(outv, out_hbm.at[pl.ds(base, _BW)])


def kernel(users, movies, user_table, movie_table, W, b):
    wb = jnp.concatenate([
        W.reshape(-1).astype(jnp.float32),
        b.astype(jnp.float32),
        jnp.zeros((15,), jnp.float32),
    ])
    out = _embed_dot(users.astype(jnp.int32), movies.astype(jnp.int32),
                     user_table, movie_table, wb)
    return out.reshape(_B, 1)


# speedup vs baseline: 6.6017x; 6.6017x over previous
import functools

import jax
import jax.numpy as jnp
from jax import lax
from jax.experimental import pallas as pl
from jax.experimental.pallas import tpu as pltpu
from jax.experimental.pallas import tpu_sc as plsc

_B = 16384
_D = 32
_NC = 2
_NS = 16
_NW = _NC * _NS
_BW = _B // _NW
_CH = 128
_NCH = _BW // _CH

_MV_BLK = 32768


def _mv_body(t_ref, w_ref, b_ref, o_ref):
    o_ref[...] = jnp.sum(t_ref[...] * w_ref[...], axis=0) + b_ref[0]


def _matvec(tt, w_col, bias_val):
    n = tt.shape[1]
    return pl.pallas_call(
        _mv_body,
        grid=(pl.cdiv(n, _MV_BLK),),
        in_specs=[
            pl.BlockSpec((_D, _MV_BLK), lambda i: (0, i)),
            pl.BlockSpec((_D, 1), lambda i: (0, 0)),
            pl.BlockSpec(memory_space=pltpu.SMEM),
        ],
        out_specs=pl.BlockSpec((_MV_BLK,), lambda i: (i,)),
        out_shape=jax.ShapeDtypeStruct((n,), jnp.float32),
    )(tt, w_col, bias_val)


_sc_mesh = plsc.VectorSubcoreMesh(core_axis_name="c", subcore_axis_name="s")


@functools.partial(
    pl.kernel,
    mesh=_sc_mesh,
    compiler_params=pltpu.CompilerParams(needs_layout_passes=False),
    out_type=jax.ShapeDtypeStruct((_B,), jnp.float32),
    scratch_types=[
        pltpu.VMEM((_BW,), jnp.int32),
        pltpu.VMEM((_BW,), jnp.int32),
        pltpu.VMEM((_BW,), jnp.float32),
        pltpu.VMEM((_BW,), jnp.float32),
        pltpu.VMEM((_BW,), jnp.float32),
        pltpu.SemaphoreType.DMA,
    ],
)
def _gather_combine(su_hbm, sm_hbm, users_hbm, movies_hbm, out_hbm,
                    uidx, midx, su, sm, outv, sem):
    wid = lax.axis_index("s") * _NC + lax.axis_index("c")
    base = wid * _BW

    pltpu.sync_copy(users_hbm.at[pl.ds(base, _BW)], uidx)
    pltpu.sync_copy(movies_hbm.at[pl.ds(base, _BW)], midx)

    copies = []
    for c in range(_NCH):
        sl = pl.ds(c * _CH, _CH)
        copies.append(pltpu.async_copy(su_hbm.at[uidx.at[sl]], su.at[sl], sem))
        copies.append(pltpu.async_copy(sm_hbm.at[midx.at[sl]], sm.at[sl], sem))
    for cp in copies:
        cp.wait()

    for g in range(_BW // 16):
        sl = pl.ds(g * 16, 16)
        outv[sl] = su[sl] + sm[sl]

    pltpu.sync_copy(outv, out_hbm.at[pl.ds(base, _BW)])


def kernel(users, movies, user_table, movie_table, W, b):
    w = W.reshape(-1).astype(jnp.float32)
    scores_u = _matvec(user_table.T, w[:_D].reshape(_D, 1),
                       jnp.zeros((1,), jnp.float32))
    scores_m = _matvec(movie_table.T, w[_D:].reshape(_D, 1),
                       b.astype(jnp.float32).reshape(1))
    out = _gather_combine(scores_u, scores_m,
                          users.astype(jnp.int32), movies.astype(jnp.int32))
    return out.reshape(_B, 1)


# speedup vs baseline: 7.8024x; 1.1819x over previous
import functools

import jax
import jax.numpy as jnp
from jax import lax
from jax.experimental import pallas as pl
from jax.experimental.pallas import tpu as pltpu
from jax.experimental.pallas import tpu_sc as plsc

_B = 16384
_D = 32
_NC = 2
_NS = 16
_NW = _NC * _NS
_BW = _B // _NW
_CH = 128
_NCH = _BW // _CH

_NU = 1000000
_NM = 100000
_GRID = 16
_UBLK = 65536
_MBLK = 8192
_MLAST = 12


def _mv_body(ut_ref, mt_ref, w_ref, b_ref, su_ref, sm_ref):
    su_ref[...] = jnp.sum(ut_ref[...] * w_ref[:_D, 0:1], axis=0)
    sm_ref[...] = jnp.sum(mt_ref[...] * w_ref[_D:, 0:1], axis=0) + b_ref[0]


def _matvecs(ut, mt, w_col, bias_val):
    return pl.pallas_call(
        _mv_body,
        grid=(_GRID,),
        in_specs=[
            pl.BlockSpec((_D, _UBLK), lambda i: (0, i)),
            pl.BlockSpec((_D, _MBLK), lambda i: (0, jnp.minimum(i, _MLAST))),
            pl.BlockSpec((2 * _D, 1), lambda i: (0, 0)),
            pl.BlockSpec(memory_space=pltpu.SMEM),
        ],
        out_specs=[
            pl.BlockSpec((_UBLK,), lambda i: (i,)),
            pl.BlockSpec((_MBLK,), lambda i: (jnp.minimum(i, _MLAST),)),
        ],
        out_shape=[
            jax.ShapeDtypeStruct((_NU,), jnp.float32),
            jax.ShapeDtypeStruct((_NM,), jnp.float32),
        ],
    )(ut, mt, w_col, bias_val)


_sc_mesh = plsc.VectorSubcoreMesh(core_axis_name="c", subcore_axis_name="s")


@functools.partial(
    pl.kernel,
    mesh=_sc_mesh,
    compiler_params=pltpu.CompilerParams(needs_layout_passes=False),
    out_type=jax.ShapeDtypeStruct((_B,), jnp.float32),
    scratch_types=[
        pltpu.VMEM((_BW,), jnp.int32),
        pltpu.VMEM((_BW,), jnp.int32),
        pltpu.VMEM((_BW,), jnp.float32),
        pltpu.VMEM((_BW,), jnp.float32),
        pltpu.VMEM((_BW,), jnp.float32),
        pltpu.SemaphoreType.DMA,
    ],
)
def _gather_combine(su_hbm, sm_hbm, users_hbm, movies_hbm, out_hbm,
                    uidx, midx, su, sm, outv, sem):
    wid = lax.axis_index("s") * _NC + lax.axis_index("c")
    base = wid * _BW

    pltpu.sync_copy(users_hbm.at[pl.ds(base, _BW)], uidx)
    pltpu.sync_copy(movies_hbm.at[pl.ds(base, _BW)], midx)

    copies = []
    for c in range(_NCH):
        sl = pl.ds(c * _CH, _CH)
        copies.append(pltpu.async_copy(su_hbm.at[uidx.at[sl]], su.at[sl], sem))
        copies.append(pltpu.async_copy(sm_hbm.at[midx.at[sl]], sm.at[sl], sem))
    for cp in copies:
        cp.wait()

    for g in range(_BW // 16):
        sl = pl.ds(g * 16, 16)
        outv[sl] = su[sl] + sm[sl]

    pltpu.sync_copy(outv, out_hbm.at[pl.ds(base, _BW)])


def kernel(users, movies, user_table, movie_table, W, b):
    w_col = W.reshape(2 * _D, 1).astype(jnp.float32)
    scores_u, scores_m = _matvecs(user_table.T, movie_table.T, w_col,
                                  b.astype(jnp.float32).reshape(1))
    out = _gather_combine(scores_u, scores_m,
                          users.astype(jnp.int32), movies.astype(jnp.int32))
    return out.reshape(_B, 1)
